# baseline (speedup 1.0000x reference)
import functools

import jax
import jax.numpy as jnp
from jax import lax
from jax.experimental import pallas as pl
from jax.experimental.pallas import tpu as pltpu
from jax.experimental.pallas import tpu_sc as plsc

NC = 2
NS = 16
NW = NC * NS
LANES = 16


def _sc_embed(x_flat, token_table, pos_table):
    flat, = x_flat.shape
    _, emb = token_table.shape
    t_len, _ = pos_table.shape

    per_w = flat // NW
    chunk = 2 * t_len
    n_chunk = per_w // chunk
    vecs_per_row = emb // LANES

    sub_sizes = []
    off = 0
    while off < chunk:
        n = min(128, chunk - off)
        sub_sizes.append((off, n))
        off += n

    mesh = plsc.VectorSubcoreMesh(core_axis_name="c", subcore_axis_name="s")

    @functools.partial(
        pl.kernel,
        mesh=mesh,
        compiler_params=pltpu.CompilerParams(use_tc_tiling_on_sc=False),
        out_type=jax.ShapeDtypeStruct((flat, emb), jnp.float32),
        scratch_types=[
            pltpu.VMEM((per_w,), jnp.int32),
            pltpu.VMEM((chunk, emb), jnp.float32),
            pltpu.VMEM((chunk, emb), jnp.float32),
            pltpu.SemaphoreType.DMA,
        ],
    )
    def k(idx_hbm, tok_hbm, pos_hbm, out_hbm, idx_v, pos_v, buf, sem):
        wid = lax.axis_index("s") * NC + lax.axis_index("c")
        base = wid * per_w
        pltpu.sync_copy(idx_hbm.at[pl.ds(base, per_w)], idx_v)
        pltpu.sync_copy(pos_hbm, pos_v.at[pl.ds(0, t_len)])
        pltpu.sync_copy(pos_hbm, pos_v.at[pl.ds(t_len, t_len)])

        def chunk_body(g, carry):
            off = pl.multiple_of(g * chunk, 8)
            for (so, sn) in sub_sizes:
                pltpu.async_copy(
                    tok_hbm.at[idx_v.at[pl.ds(off + so, sn)]],
                    buf.at[pl.ds(so, sn)],
                    sem,
                ).wait()

            def row_body(j, c2):
                for v in range(vecs_per_row):
                    pv = pos_v[j, pl.ds(v * LANES, LANES)]
                    plsc.addupdate(buf.at[j, pl.ds(v * LANES, LANES)], pv)
                return c2

            lax.fori_loop(0, chunk, row_body, 0, unroll=4)
            pltpu.sync_copy(buf, out_hbm.at[pl.ds(base + off, chunk)])
            return carry

        lax.fori_loop(0, n_chunk, chunk_body, 0)

    return k(x_flat, token_table, pos_table)


def kernel(x, token_table, pos_table):
    b, t = x.shape
    _, emb = token_table.shape
    x_flat = x.reshape(b * t).astype(jnp.int32)
    out = _sc_embed(x_flat, token_table, pos_table)
    return out.reshape(b, t, emb)


# speedup vs baseline: 1.1976x; 1.1976x over previous
import functools

import jax
import jax.numpy as jnp
from jax import lax
from jax.experimental import pallas as pl
from jax.experimental.pallas import tpu as pltpu
from jax.experimental.pallas import tpu_sc as plsc

NC = 2
NS = 16
NW = NC * NS
LANES = 16
NBUF = 4


def _sc_embed(x_flat, token_table, pos_table):
    flat, = x_flat.shape
    _, emb = token_table.shape
    t_len, _ = pos_table.shape

    per_w = flat // NW
    chunk = t_len
    n_chunk = per_w // chunk
    vecs_per_row = emb // LANES

    sub_sizes = []
    off = 0
    while off < chunk:
        n = min(128, chunk - off)
        sub_sizes.append((off, n))
        off += n

    mesh = plsc.VectorSubcoreMesh(core_axis_name="c", subcore_axis_name="s")

    @functools.partial(
        pl.kernel,
        mesh=mesh,
        compiler_params=pltpu.CompilerParams(use_tc_tiling_on_sc=False),
        out_type=jax.ShapeDtypeStruct((flat, emb), jnp.float32),
        scratch_types=[
            pltpu.VMEM((per_w,), jnp.int32),
            pltpu.VMEM((t_len, emb), jnp.float32),
            [pltpu.VMEM((chunk, emb), jnp.float32) for _ in range(NBUF)],
            [pltpu.SemaphoreType.DMA for _ in range(NBUF)],
            [pltpu.SemaphoreType.DMA for _ in range(NBUF)],
        ],
    )
    def k(idx_hbm, tok_hbm, pos_hbm, out_hbm, idx_v, pos_v, bufs, gsems, ssems):
        wid = lax.axis_index("s") * NC + lax.axis_index("c")
        base = wid * per_w
        pltpu.sync_copy(idx_hbm.at[pl.ds(base, per_w)], idx_v)
        pltpu.sync_copy(pos_hbm, pos_v)

        def g_issue(off, buf, sem):
            for so, sn in sub_sizes:
                pltpu.async_copy(
                    tok_hbm.at[idx_v.at[pl.ds(off + so, sn)]],
                    buf.at[pl.ds(so, sn)], sem)

        def g_wait(off, buf, sem):
            for so, sn in sub_sizes:
                pltpu.make_async_copy(
                    tok_hbm.at[idx_v.at[pl.ds(off + so, sn)]],
                    buf.at[pl.ds(so, sn)], sem).wait()

        def s_issue(off, buf, sem):
            pltpu.async_copy(buf, out_hbm.at[pl.ds(base + off, chunk)], sem)

        def s_wait(off, buf, sem):
            pltpu.make_async_copy(
                buf, out_hbm.at[pl.ds(base + off, chunk)], sem).wait()

        def add_pos(buf):
            @plsc.parallel_loop(0, chunk, 1, unroll=8)
            def _(j):
                for v in range(vecs_per_row):
                    sl = pl.ds(v * LANES, LANES)
                    plsc.addupdate(buf.at[j, sl], pos_v[j, sl])

        g_issue(0, bufs[0], gsems[0])
        g_issue(chunk, bufs[1], gsems[1])

        def macro(m, carry):
            for i in range(NBUF):
                g = NBUF * m + i
                off = pl.multiple_of(g * chunk, 8)
                g_wait(off, bufs[i], gsems[i])
                add_pos(bufs[i])
                s_issue(off, bufs[i], ssems[i])
                i2 = (i + 2) % NBUF
                g2 = g + 2
                off2 = pl.multiple_of(g2 * chunk, 8)

                @pl.when(g2 < n_chunk)
                def _():
                    @pl.when(g2 >= NBUF)
                    def _():
                        s_wait(pl.multiple_of((g - 2) * chunk, 8),
                               bufs[i2], ssems[i2])
                    g_issue(off2, bufs[i2], gsems[i2])
            return carry

        lax.fori_loop(0, n_chunk // NBUF, macro, 0)

        for i in range(NBUF):
            s_wait((n_chunk - NBUF + i) * chunk, bufs[i], ssems[i])

    return k(x_flat, token_table, pos_table)


def kernel(x, token_table, pos_table):
    b, t = x.shape
    _, emb = token_table.shape
    x_flat = x.reshape(b * t).astype(jnp.int32)
    out = _sc_embed(x_flat, token_table, pos_table)
    return out.reshape(b, t, emb)
